# Initial kernel scaffold; baseline (speedup 1.0000x reference)
import jax
import jax.numpy as jnp
from jax.experimental import pallas as pl


def kernel(cls_preds, bbox_preds, anchors, labels, boxes):
    raise NotImplementedError("write your pallas kernel here")


# speedup vs baseline: 4.9944x; 4.9944x over previous
import functools

import jax
import jax.numpy as jnp
import numpy as np
from jax.experimental import pallas as pl
from jax.experimental.pallas import tpu as pltpu

NUM_CLASSES = 80
B = 4
N = 120000
G = 64
PER = G + 2
LANES = 128

CLS_ROWS = N * NUM_CLASSES // LANES
CLS_BR = 1320
CLS_J = -(-CLS_ROWS // CLS_BR)

BOX_ROWS = N * 4 // LANES
BOX_BR = 264
BOX_J = -(-BOX_ROWS // BOX_BR)

_FULL = N // PER
_TAIL = N - _FULL * PER
S_MATCHED = _FULL * G + max(0, _TAIL - 2)


def _np_masks():
    f_cls = np.arange(CLS_BR * LANES)
    m_cls = ((f_cls // NUM_CLASSES) % PER != 0).astype(np.float32)
    f_box = np.arange(BOX_BR * LANES)
    m_box = ((f_box // 4) % PER >= 2).astype(np.float32)
    return (m_cls.reshape(CLS_BR, LANES), m_box.reshape(BOX_BR, LANES))


_MCLS_NP, _MBOX_NP = _np_masks()


def _loss_body(cls_ref, tcls_ref, mcls_ref, box_ref, anc_ref, btile_ref,
               mbox_ref, out_ref, acc_ref):
    b = pl.program_id(0)
    j = pl.program_id(1)

    @pl.when((b == 0) & (j == 0))
    def _init():
        acc_ref[0] = 0.0
        acc_ref[1] = 0.0

    x = cls_ref[0]
    t = tcls_ref[0]
    m = mcls_ref[...]
    e = jnp.exp(-jnp.abs(x))
    lg = jnp.log1p(e)
    inv = 1.0 / (1.0 + e)
    ps = jnp.where(x >= 0.0, inv, e * inv)
    relux = jnp.maximum(x, 0.0)
    bce0 = relux + lg
    bce1 = relux - x + lg
    omp = 1.0 - ps
    f0 = (0.25 * ps * ps) * bce0
    f1 = (0.75 * omp * omp) * bce1
    base = f0 + t * (f1 - f0)

    @pl.when(j < CLS_J - 1)
    def _cls_full():
        acc_ref[0] += jnp.sum(jnp.where(m > 0.5, base, 0.0))

    @pl.when(j == CLS_J - 1)
    def _cls_tail():
        row = jax.lax.broadcasted_iota(jnp.int32, (CLS_BR, LANES), 0)
        ok = (m > 0.5) & (row + j * CLS_BR < CLS_ROWS)
        acc_ref[0] += jnp.sum(jnp.where(ok, base, 0.0))

    @pl.when(j < BOX_J)
    def _box():
        bp = box_ref[0]
        a = anc_ref[0]
        bt_tab = btile_ref[0]
        mb = mbox_ref[...]
        comp = jax.lax.broadcasted_iota(jnp.int32, (BOX_BR, LANES), 1)
        iscen = (comp & 3) < 2
        rolled = pltpu.roll(a, shift=LANES - 2, axis=1)
        wh = jnp.where(iscen, rolled, a)
        bt_c = (bt_tab - a) / wh * 10.0
        bt_s = jnp.log(bt_tab / wh + 1e-8) * 5.0
        bt = jnp.where(iscen, bt_c, bt_s)
        d = jnp.abs(bp - bt)
        hub = jnp.where(d < 1.0, 0.5 * d * d, d - 0.5)
        ok = mb > 0.5
        if True:
            row = jax.lax.broadcasted_iota(jnp.int32, (BOX_BR, LANES), 0)
            ok = ok & (row + j * BOX_BR < BOX_ROWS)
        acc_ref[1] += jnp.sum(jnp.where(ok, hub, 0.0))

    @pl.when((b == B - 1) & (j == CLS_J - 1))
    def _fin():
        out_ref[0] = acc_ref[0] / np.float32(S_MATCHED * B)
        out_ref[1] = acc_ref[1] / np.float32(S_MATCHED * 4 * B)


@functools.partial(jax.jit, static_argnames=("interpret",))
def _run(cls_preds, bbox_preds, anchors, labels, boxes, interpret=False):
    cls_r = cls_preds.reshape(B, CLS_ROWS, LANES)
    box_r = bbox_preds.reshape(B, BOX_ROWS, LANES)
    anc_r = anchors.reshape(B, BOX_ROWS, LANES)

    oh = jax.nn.one_hot(labels, NUM_CLASSES, dtype=jnp.float32)
    oh = jnp.concatenate([jnp.zeros((B, 2, NUM_CLASSES), jnp.float32), oh], 1)
    oh = oh.reshape(B, 1, PER * NUM_CLASSES)
    tcls = jnp.tile(oh, (1, CLS_BR * LANES // (PER * NUM_CLASSES), 1))
    tcls = tcls.reshape(B, CLS_BR, LANES)

    bx = jnp.concatenate([jnp.ones((B, 2, 4), jnp.float32), boxes], 1)
    bx = bx.reshape(B, 1, PER * 4)
    btile = jnp.tile(bx, (1, BOX_BR * LANES // (PER * 4), 1))
    btile = btile.reshape(B, BOX_BR, LANES)

    mcls = jnp.asarray(_MCLS_NP)
    mbox = jnp.asarray(_MBOX_NP)

    out = pl.pallas_call(
        _loss_body,
        grid=(B, CLS_J),
        in_specs=[
            pl.BlockSpec((1, CLS_BR, LANES), lambda b, j: (b, j, 0)),
            pl.BlockSpec((1, CLS_BR, LANES), lambda b, j: (b, 0, 0)),
            pl.BlockSpec((CLS_BR, LANES), lambda b, j: (0, 0)),
            pl.BlockSpec((1, BOX_BR, LANES),
                         lambda b, j: (b, jnp.minimum(j, BOX_J - 1), 0)),
            pl.BlockSpec((1, BOX_BR, LANES),
                         lambda b, j: (b, jnp.minimum(j, BOX_J - 1), 0)),
            pl.BlockSpec((1, BOX_BR, LANES), lambda b, j: (b, 0, 0)),
            pl.BlockSpec((BOX_BR, LANES), lambda b, j: (0, 0)),
        ],
        out_specs=pl.BlockSpec(memory_space=pltpu.SMEM),
        out_shape=jax.ShapeDtypeStruct((2,), jnp.float32),
        scratch_shapes=[pltpu.SMEM((2,), jnp.float32)],
        compiler_params=pltpu.CompilerParams(
            dimension_semantics=("arbitrary", "arbitrary")),
        interpret=interpret,
    )(cls_r, tcls, mcls, box_r, anc_r, btile, mbox)
    return out


def kernel(cls_preds, bbox_preds, anchors, labels, boxes):
    return _run(cls_preds, bbox_preds, anchors, labels, boxes)


# speedup vs baseline: 5.6172x; 1.1247x over previous
import functools

import jax
import jax.numpy as jnp
import numpy as np
from jax.experimental import pallas as pl
from jax.experimental.pallas import tpu as pltpu

NUM_CLASSES = 80
B = 4
N = 120000
G = 64
PER = G + 2
LANES = 128

CLS_TR = 8 * PER
CLS_BR = 10 * CLS_TR
CLS_J = -(-N // CLS_BR)

BOX_ROWS = N * 4 // LANES
BOX_BR = 264
BOX_J = -(-BOX_ROWS // BOX_BR)

_FULL = N // PER
_TAIL = N - _FULL * PER
S_MATCHED = _FULL * G + max(0, _TAIL - 2)


def _np_masks():
    a_cls = np.arange(CLS_TR)
    m_cls = np.broadcast_to(((a_cls % PER) != 0)[:, None],
                            (CLS_TR, NUM_CLASSES)).astype(np.float32)
    f_box = np.arange(BOX_BR * LANES)
    m_box = ((f_box // 4) % PER >= 2).astype(np.float32)
    return m_cls.copy(), m_box.reshape(BOX_BR, LANES)


_MCLS_NP, _MBOX_NP = _np_masks()


def _loss_body(cls_ref, tcls_ref, mcls_ref, box_ref, anc_ref, btile_ref,
               mbox_ref, out_ref, acc_ref):
    b = pl.program_id(0)
    j = pl.program_id(1)

    @pl.when((b == 0) & (j == 0))
    def _init():
        acc_ref[0] = 0.0
        acc_ref[1] = 0.0

    x = cls_ref[0]
    t = jnp.tile(tcls_ref[0], (CLS_BR // CLS_TR, 1))
    m = jnp.tile(mcls_ref[...], (CLS_BR // CLS_TR, 1))
    e = jnp.exp(-jnp.abs(x))
    lg = jnp.log1p(e)
    inv = 1.0 / (1.0 + e)
    ps = jnp.where(x >= 0.0, inv, e * inv)
    relux = jnp.maximum(x, 0.0)
    bce0 = relux + lg
    bce1 = relux - x + lg
    omp = 1.0 - ps
    f0 = (0.25 * ps * ps) * bce0
    f1 = (0.75 * omp * omp) * bce1
    base = f0 + t * (f1 - f0)

    @pl.when(j < CLS_J - 1)
    def _cls_full():
        acc_ref[0] += jnp.sum(jnp.where(m > 0.5, base, 0.0))

    @pl.when(j == CLS_J - 1)
    def _cls_tail():
        row = jax.lax.broadcasted_iota(jnp.int32, (CLS_BR, NUM_CLASSES), 0)
        ok = (m > 0.5) & (row + j * CLS_BR < N)
        acc_ref[0] += jnp.sum(jnp.where(ok, base, 0.0))

    @pl.when(j < BOX_J)
    def _box():
        bp = box_ref[0]
        a = anc_ref[0]
        bt_tab = btile_ref[0]
        mb = mbox_ref[...]
        comp = jax.lax.broadcasted_iota(jnp.int32, (BOX_BR, LANES), 1)
        iscen = (comp & 3) < 2
        rolled = pltpu.roll(a, shift=LANES - 2, axis=1)
        wh = jnp.where(iscen, rolled, a)
        bt_c = (bt_tab - a) / wh * 10.0
        bt_s = jnp.log(bt_tab / wh + 1e-8) * 5.0
        bt = jnp.where(iscen, bt_c, bt_s)
        d = jnp.abs(bp - bt)
        hub = jnp.where(d < 1.0, 0.5 * d * d, d - 0.5)
        row = jax.lax.broadcasted_iota(jnp.int32, (BOX_BR, LANES), 0)
        ok = (mb > 0.5) & (row + j * BOX_BR < BOX_ROWS)
        acc_ref[1] += jnp.sum(jnp.where(ok, hub, 0.0))

    @pl.when((b == B - 1) & (j == CLS_J - 1))
    def _fin():
        out_ref[0] = acc_ref[0] / np.float32(S_MATCHED * B)
        out_ref[1] = acc_ref[1] / np.float32(S_MATCHED * 4 * B)


@functools.partial(jax.jit, static_argnames=("interpret",))
def _run(cls_preds, bbox_preds, anchors, labels, boxes, interpret=False):
    box_r = bbox_preds.reshape(B, BOX_ROWS, LANES)
    anc_r = anchors.reshape(B, BOX_ROWS, LANES)

    oh = jax.nn.one_hot(labels, NUM_CLASSES, dtype=jnp.float32)
    oh = jnp.concatenate([jnp.zeros((B, 2, NUM_CLASSES), jnp.float32), oh], 1)
    tcls = jnp.tile(oh, (1, CLS_TR // PER, 1))

    bx = jnp.concatenate([jnp.ones((B, 2, 4), jnp.float32), boxes], 1)
    bx = bx.reshape(B, 1, PER * 4)
    btile = jnp.tile(bx, (1, BOX_BR * LANES // (PER * 4), 1))
    btile = btile.reshape(B, BOX_BR, LANES)

    mcls = jnp.asarray(_MCLS_NP)
    mbox = jnp.asarray(_MBOX_NP)

    out = pl.pallas_call(
        _loss_body,
        grid=(B, CLS_J),
        in_specs=[
            pl.BlockSpec((1, CLS_BR, NUM_CLASSES), lambda b, j: (b, j, 0)),
            pl.BlockSpec((1, CLS_TR, NUM_CLASSES), lambda b, j: (b, 0, 0)),
            pl.BlockSpec((CLS_TR, NUM_CLASSES), lambda b, j: (0, 0)),
            pl.BlockSpec((1, BOX_BR, LANES),
                         lambda b, j: (b, jnp.minimum(j, BOX_J - 1), 0)),
            pl.BlockSpec((1, BOX_BR, LANES),
                         lambda b, j: (b, jnp.minimum(j, BOX_J - 1), 0)),
            pl.BlockSpec((1, BOX_BR, LANES), lambda b, j: (b, 0, 0)),
            pl.BlockSpec((BOX_BR, LANES), lambda b, j: (0, 0)),
        ],
        out_specs=pl.BlockSpec(memory_space=pltpu.SMEM),
        out_shape=jax.ShapeDtypeStruct((2,), jnp.float32),
        scratch_shapes=[pltpu.SMEM((2,), jnp.float32)],
        compiler_params=pltpu.CompilerParams(
            dimension_semantics=("arbitrary", "arbitrary")),
        interpret=interpret,
    )(cls_preds, tcls, mcls, box_r, anc_r, btile, mbox)
    return out


def kernel(cls_preds, bbox_preds, anchors, labels, boxes):
    return _run(cls_preds, bbox_preds, anchors, labels, boxes)


# speedup vs baseline: 8.3008x; 1.4777x over previous
import functools

import jax
import jax.numpy as jnp
import numpy as np
from jax.experimental import pallas as pl
from jax.experimental.pallas import tpu as pltpu

NUM_CLASSES = 80
B = 4
N = 120000
G = 64
PER = G + 2
LANES = 128

CLS_TR = 8 * PER
CLS_BR = 10 * CLS_TR
CLS_J = -(-N // CLS_BR)

BOX_TR = 8 * PER
BOX_BR = 16 * BOX_TR
BOX_J = -(-N // BOX_BR)

_FULL = N // PER
_TAIL = N - _FULL * PER
S_MATCHED = _FULL * G + max(0, _TAIL - 2)


def _np_masks():
    a_cls = np.arange(CLS_TR)
    m_cls = np.broadcast_to(((a_cls % PER) != 0)[:, None],
                            (CLS_TR, NUM_CLASSES)).astype(np.float32)
    a_box = np.arange(BOX_TR)
    m_box = np.broadcast_to(((a_box % PER) >= 2)[:, None],
                            (BOX_TR, 4)).astype(np.float32)
    return m_cls.copy(), m_box.copy()


_MCLS_NP, _MBOX_NP = _np_masks()


def _loss_body(cls_ref, tcls_ref, mcls_ref, box_ref, anc_ref, btile_ref,
               mbox_ref, out_ref, acc_ref):
    b = pl.program_id(0)
    j = pl.program_id(1)

    @pl.when((b == 0) & (j == 0))
    def _init():
        acc_ref[0] = 0.0
        acc_ref[1] = 0.0

    x = cls_ref[0]
    t = jnp.tile(tcls_ref[0], (CLS_BR // CLS_TR, 1))
    m = jnp.tile(mcls_ref[...], (CLS_BR // CLS_TR, 1))
    e = jnp.exp(-jnp.abs(x))
    lg = jnp.log1p(e)
    inv = 1.0 / (1.0 + e)
    ps = jnp.where(x >= 0.0, inv, e * inv)
    relux = jnp.maximum(x, 0.0)
    bce0 = relux + lg
    bce1 = relux - x + lg
    omp = 1.0 - ps
    f0 = (0.25 * ps * ps) * bce0
    f1 = (0.75 * omp * omp) * bce1
    base = f0 + t * (f1 - f0)

    @pl.when(j < CLS_J - 1)
    def _cls_full():
        acc_ref[0] += jnp.sum(jnp.where(m > 0.5, base, 0.0))

    @pl.when(j == CLS_J - 1)
    def _cls_tail():
        row = jax.lax.broadcasted_iota(jnp.int32, (CLS_BR, NUM_CLASSES), 0)
        ok = (m > 0.5) & (row + j * CLS_BR < N)
        acc_ref[0] += jnp.sum(jnp.where(ok, base, 0.0))

    @pl.when(j < BOX_J)
    def _box():
        bp = box_ref[0]
        a = anc_ref[0]
        bt_tab = jnp.tile(btile_ref[0], (BOX_BR // BOX_TR, 1))
        mb = jnp.tile(mbox_ref[...], (BOX_BR // BOX_TR, 1))
        comp = jax.lax.broadcasted_iota(jnp.int32, (BOX_BR, 4), 1)
        iscen = comp < 2
        rolled = pltpu.roll(a, shift=2, axis=1)
        wh = jnp.where(iscen, rolled, a)
        bt_c = (bt_tab - a) / wh * 10.0
        bt_s = jnp.log(bt_tab / wh + 1e-8) * 5.0
        bt = jnp.where(iscen, bt_c, bt_s)
        d = jnp.abs(bp - bt)
        hub = jnp.where(d < 1.0, 0.5 * d * d, d - 0.5)
        row = jax.lax.broadcasted_iota(jnp.int32, (BOX_BR, 4), 0)
        ok = (mb > 0.5) & (row + j * BOX_BR < N)
        acc_ref[1] += jnp.sum(jnp.where(ok, hub, 0.0))

    @pl.when((b == B - 1) & (j == CLS_J - 1))
    def _fin():
        out_ref[0] = acc_ref[0] / np.float32(S_MATCHED * B)
        out_ref[1] = acc_ref[1] / np.float32(S_MATCHED * 4 * B)


@functools.partial(jax.jit, static_argnames=("interpret",))
def _run(cls_preds, bbox_preds, anchors, labels, boxes, interpret=False):
    oh = jax.nn.one_hot(labels, NUM_CLASSES, dtype=jnp.float32)
    oh = jnp.concatenate([jnp.zeros((B, 2, NUM_CLASSES), jnp.float32), oh], 1)
    tcls = jnp.tile(oh, (1, CLS_TR // PER, 1))

    bx = jnp.concatenate([jnp.ones((B, 2, 4), jnp.float32), boxes], 1)
    btile = jnp.tile(bx, (1, BOX_TR // PER, 1))

    mcls = jnp.asarray(_MCLS_NP)
    mbox = jnp.asarray(_MBOX_NP)

    out = pl.pallas_call(
        _loss_body,
        grid=(B, CLS_J),
        in_specs=[
            pl.BlockSpec((1, CLS_BR, NUM_CLASSES), lambda b, j: (b, j, 0)),
            pl.BlockSpec((1, CLS_TR, NUM_CLASSES), lambda b, j: (b, 0, 0)),
            pl.BlockSpec((CLS_TR, NUM_CLASSES), lambda b, j: (0, 0)),
            pl.BlockSpec((1, BOX_BR, 4),
                         lambda b, j: (b, jnp.minimum(j, BOX_J - 1), 0)),
            pl.BlockSpec((1, BOX_BR, 4),
                         lambda b, j: (b, jnp.minimum(j, BOX_J - 1), 0)),
            pl.BlockSpec((1, BOX_TR, 4), lambda b, j: (b, 0, 0)),
            pl.BlockSpec((BOX_TR, 4), lambda b, j: (0, 0)),
        ],
        out_specs=pl.BlockSpec(memory_space=pltpu.SMEM),
        out_shape=jax.ShapeDtypeStruct((2,), jnp.float32),
        scratch_shapes=[pltpu.SMEM((2,), jnp.float32)],
        compiler_params=pltpu.CompilerParams(
            dimension_semantics=("arbitrary", "arbitrary")),
        interpret=interpret,
    )(cls_preds, tcls, mcls, bbox_preds, anchors, btile, mbox)
    return out


def kernel(cls_preds, bbox_preds, anchors, labels, boxes):
    return _run(cls_preds, bbox_preds, anchors, labels, boxes)
